# Initial kernel scaffold; baseline (speedup 1.0000x reference)
import functools

import jax
import jax.numpy as jnp
from jax import lax
from jax.experimental import pallas as pl
from jax.experimental.pallas import tpu as pltpu
from jax.experimental.pallas import tpu_sc as plsc

B = 4096
NF = 26
HIST = 200
DIM = 32
SV = 100000
IV = 1000000
KV_DIM = 8
EPS = 1e-6

NC = 2
NS = 16
NW = NC * NS

S_ROWS = B * NF
H_ROWS = B * HIST
S_CHUNKS = S_ROWS // (NW * 128)
H_CHUNKS = H_ROWS // (NW * 128)
K = 8
ROWS_BUF = K * 128


def _sc_gather(sidx, hidx, stab, itab):
    mesh = plsc.VectorSubcoreMesh(core_axis_name="c", subcore_axis_name="s")

    @functools.partial(
        pl.kernel,
        mesh=mesh,
        out_type=(
            jax.ShapeDtypeStruct((S_ROWS, DIM), jnp.float32),
            jax.ShapeDtypeStruct((H_ROWS, DIM), jnp.float32),
        ),
        scratch_types=[
            pltpu.VMEM((S_CHUNKS, 128), jnp.int32),
            pltpu.VMEM((H_CHUNKS, 128), jnp.int32),
            pltpu.VMEM((ROWS_BUF, DIM), jnp.float32),
            pltpu.SemaphoreType.DMA,
        ],
    )
    def gather_k(sidx_hbm, hidx_hbm, stab_hbm, itab_hbm, sctx_hbm, hctx_hbm,
                 sidx_v, hidx_v, rows_v, sem):
        wid = lax.axis_index("s") * NC + lax.axis_index("c")
        pltpu.sync_copy(sidx_hbm.at[pl.ds(wid * S_CHUNKS, S_CHUNKS)], sidx_v)
        pltpu.sync_copy(hidx_hbm.at[pl.ds(wid * H_CHUNKS, H_CHUNKS)], hidx_v)

        def do_group(idx_v, g0, k, src_tab, out_hbm, out_base):
            cps = []
            for j in range(k):
                cp = pltpu.async_copy(
                    src_tab.at[idx_v.at[g0 + j]],
                    rows_v.at[pl.ds(j * 128, 128)],
                    sem,
                )
                cps.append(cp)
            for cp in cps:
                cp.wait()
            pltpu.sync_copy(rows_v.at[pl.ds(0, k * 128)],
                            out_hbm.at[pl.ds(out_base, k * 128)])

        s_base = wid * (S_CHUNKS * 128)
        h_base = wid * (H_CHUNKS * 128)

        def sbody(g, carry):
            do_group(sidx_v, g * K, K, stab_hbm, sctx_hbm,
                     s_base + g * ROWS_BUF)
            return carry

        lax.fori_loop(0, S_CHUNKS // K, sbody, 0)
        do_group(sidx_v, (S_CHUNKS // K) * K, S_CHUNKS % K, stab_hbm,
                 sctx_hbm, s_base + (S_CHUNKS // K) * ROWS_BUF)

        def hbody(g, carry):
            do_group(hidx_v, g * K, K, itab_hbm, hctx_hbm,
                     h_base + g * ROWS_BUF)
            return carry

        lax.fori_loop(0, H_CHUNKS // K, hbody, 0)

    return gather_k(sidx, hidx, stab, itab)


def _tc_proj_body(s_ref, h_ref, w_ref, b_ref, nw_ref, outs_ref, outh_ref):
    w = w_ref[...]
    bias = b_ref[...]
    nw = nw_ref[...]

    def proj(x):
        p = jax.lax.dot_general(
            x, w, dimension_numbers=(((1,), (0,)), ((), ())),
            preferred_element_type=jnp.float32) + bias
        ms = jnp.mean(p * p, axis=-1, keepdims=True)
        return p * jax.lax.rsqrt(ms + EPS) * nw

    outs_ref[...] = proj(s_ref[...])
    outh_ref[...] = proj(h_ref[...])


def _tc_project(sctx, hctx, w, bias, nw):
    grid = 32
    bs = S_ROWS // grid
    bh = H_ROWS // grid
    return pl.pallas_call(
        _tc_proj_body,
        grid=(grid,),
        in_specs=[
            pl.BlockSpec((bs, DIM), lambda i: (i, 0)),
            pl.BlockSpec((bh, DIM), lambda i: (i, 0)),
            pl.BlockSpec((DIM, KV_DIM), lambda i: (0, 0)),
            pl.BlockSpec((1, KV_DIM), lambda i: (0, 0)),
            pl.BlockSpec((1, KV_DIM), lambda i: (0, 0)),
        ],
        out_specs=[
            pl.BlockSpec((bs, KV_DIM), lambda i: (i, 0)),
            pl.BlockSpec((bh, KV_DIM), lambda i: (i, 0)),
        ],
        out_shape=[
            jax.ShapeDtypeStruct((S_ROWS, KV_DIM), jnp.float32),
            jax.ShapeDtypeStruct((H_ROWS, KV_DIM), jnp.float32),
        ],
    )(sctx, hctx, w, bias, nw)


def kernel(user_sparse, history_item_ids, sparse_tables, item_table,
           W_kv, b_kv, norm_w):
    offs = (jnp.arange(NF, dtype=jnp.int32) * (SV + 1))[None, :]
    sidx = (user_sparse + offs).reshape(S_ROWS // 128, 128)
    hidx = history_item_ids.reshape(H_ROWS // 128, 128)
    stab = sparse_tables.reshape(NF * (SV + 1), DIM)

    sctx, hctx = _sc_gather(sidx, hidx, stab, item_table)

    outs, outh = _tc_project(sctx, hctx, W_kv,
                             b_kv.reshape(1, KV_DIM),
                             norm_w.reshape(1, KV_DIM))

    return jnp.concatenate(
        [outs.reshape(B, NF, KV_DIM), outh.reshape(B, HIST, KV_DIM)],
        axis=1)


# speedup vs baseline: 1.7050x; 1.7050x over previous
import functools

import jax
import jax.numpy as jnp
from jax import lax
from jax.experimental import pallas as pl
from jax.experimental.pallas import tpu as pltpu
from jax.experimental.pallas import tpu_sc as plsc

B = 4096
NF = 26
HIST = 200
DIM = 32
SV = 100000
IV = 1000000
KV_DIM = 8
EPS = 1e-6

NC = 2
NS = 16
NW = NC * NS

S_ROWS = B * NF
H_ROWS = B * HIST
S_CHUNKS = S_ROWS // (NW * 128)
H_CHUNKS = H_ROWS // (NW * 128)
K = 8
ROWS_BUF = K * 128


def _sc_gather(sidx, hidx, stab, itab):
    mesh = plsc.VectorSubcoreMesh(core_axis_name="c", subcore_axis_name="s")

    @functools.partial(
        pl.kernel,
        mesh=mesh,
        compiler_params=pltpu.CompilerParams(use_tc_tiling_on_sc=False),
        out_type=(
            jax.ShapeDtypeStruct((S_ROWS, DIM), jnp.float32),
            jax.ShapeDtypeStruct((H_ROWS, DIM), jnp.float32),
        ),
        scratch_types=[
            pltpu.VMEM((S_CHUNKS, 128), jnp.int32),
            pltpu.VMEM((H_CHUNKS, 128), jnp.int32),
            pltpu.VMEM((ROWS_BUF, DIM), jnp.float32),
            pltpu.SemaphoreType.DMA,
        ],
    )
    def gather_k(sidx_hbm, hidx_hbm, stab_hbm, itab_hbm, sctx_hbm, hctx_hbm,
                 sidx_v, hidx_v, rows_v, sem):
        wid = lax.axis_index("s") * NC + lax.axis_index("c")
        pltpu.sync_copy(sidx_hbm.at[wid], sidx_v)
        pltpu.sync_copy(hidx_hbm.at[wid], hidx_v)

        def do_group(idx_v, g0, k, src_tab, out_hbm, out_base):
            cps = []
            for j in range(k):
                cp = pltpu.async_copy(
                    src_tab.at[idx_v.at[g0 + j]],
                    rows_v.at[pl.ds(j * 128, 128)],
                    sem,
                )
                cps.append(cp)
            for cp in cps:
                cp.wait()
            pltpu.sync_copy(rows_v.at[pl.ds(0, k * 128)],
                            out_hbm.at[pl.ds(out_base, k * 128)])

        s_base = wid * (S_CHUNKS * 128)
        h_base = wid * (H_CHUNKS * 128)

        def sbody(g, carry):
            do_group(sidx_v, g * K, K, stab_hbm, sctx_hbm,
                     s_base + g * ROWS_BUF)
            return carry

        lax.fori_loop(0, S_CHUNKS // K, sbody, 0)
        do_group(sidx_v, (S_CHUNKS // K) * K, S_CHUNKS % K, stab_hbm,
                 sctx_hbm, s_base + (S_CHUNKS // K) * ROWS_BUF)

        def hbody(g, carry):
            do_group(hidx_v, g * K, K, itab_hbm, hctx_hbm,
                     h_base + g * ROWS_BUF)
            return carry

        lax.fori_loop(0, H_CHUNKS // K, hbody, 0)

    return gather_k(sidx, hidx, stab, itab)


def _tc_proj_body(s_ref, h_ref, w_ref, b_ref, nw_ref, outs_ref, outh_ref):
    w = w_ref[...]
    bias = b_ref[...]
    nw = nw_ref[...]

    def proj(x):
        p = jax.lax.dot_general(
            x, w, dimension_numbers=(((1,), (0,)), ((), ())),
            preferred_element_type=jnp.float32) + bias
        ms = jnp.mean(p * p, axis=-1, keepdims=True)
        return p * jax.lax.rsqrt(ms + EPS) * nw

    outs_ref[...] = proj(s_ref[...])
    outh_ref[...] = proj(h_ref[...])


def _tc_project(sctx, hctx, w, bias, nw):
    grid = 32
    bs = S_ROWS // grid
    bh = H_ROWS // grid
    return pl.pallas_call(
        _tc_proj_body,
        grid=(grid,),
        in_specs=[
            pl.BlockSpec((bs, DIM), lambda i: (i, 0)),
            pl.BlockSpec((bh, DIM), lambda i: (i, 0)),
            pl.BlockSpec((DIM, KV_DIM), lambda i: (0, 0)),
            pl.BlockSpec((1, KV_DIM), lambda i: (0, 0)),
            pl.BlockSpec((1, KV_DIM), lambda i: (0, 0)),
        ],
        out_specs=[
            pl.BlockSpec((bs, KV_DIM), lambda i: (i, 0)),
            pl.BlockSpec((bh, KV_DIM), lambda i: (i, 0)),
        ],
        out_shape=[
            jax.ShapeDtypeStruct((S_ROWS, KV_DIM), jnp.float32),
            jax.ShapeDtypeStruct((H_ROWS, KV_DIM), jnp.float32),
        ],
    )(sctx, hctx, w, bias, nw)


def kernel(user_sparse, history_item_ids, sparse_tables, item_table,
           W_kv, b_kv, norm_w):
    offs = (jnp.arange(NF, dtype=jnp.int32) * (SV + 1))[None, :]
    sidx = (user_sparse + offs).reshape(NW, S_CHUNKS, 128)
    hidx = history_item_ids.reshape(NW, H_CHUNKS, 128)
    stab = sparse_tables.reshape(NF * (SV + 1), DIM)

    sctx, hctx = _sc_gather(sidx, hidx, stab, item_table)

    outs, outh = _tc_project(sctx, hctx, W_kv,
                             b_kv.reshape(1, KV_DIM),
                             norm_w.reshape(1, KV_DIM))

    return jnp.concatenate(
        [outs.reshape(B, NF, KV_DIM), outh.reshape(B, HIST, KV_DIM)],
        axis=1)


# speedup vs baseline: 5.1857x; 3.0415x over previous
import functools

import jax
import jax.numpy as jnp
from jax import lax
from jax.experimental import pallas as pl
from jax.experimental.pallas import tpu as pltpu
from jax.experimental.pallas import tpu_sc as plsc

B = 4096
NF = 26
HIST = 200
DIM = 32
SV = 100000
IV = 1000000
KV_DIM = 8
EPS = 1e-6

NC = 2
NS = 16
NW = NC * NS

S_ROWS = B * NF
H_ROWS = B * HIST
S_CHUNKS = S_ROWS // (NW * 128)
H_CHUNKS = H_ROWS // (NW * 128)
PACK = 128 // DIM

IV_ROWS = IV + 1
RB = 512
FB_FULL = (SV + 1) // RB
SB_FULL = NF * FB_FULL
HB_FULL = IV_ROWS // RB
PSF = 25008
PS_ROWS = NF * PSF
PH_ROWS = 250008
F_STRIDE = PSF * PACK


def _sc_repack(stab3, itab, lastrows):
    mesh = plsc.VectorSubcoreMesh(core_axis_name="c", subcore_axis_name="s")

    @functools.partial(
        pl.kernel,
        mesh=mesh,
        out_type=(
            jax.ShapeDtypeStruct((PS_ROWS, 128), jnp.float32),
            jax.ShapeDtypeStruct((PH_ROWS, 128), jnp.float32),
        ),
        scratch_types=[
            pltpu.VMEM((RB, DIM), jnp.float32),
            pltpu.VMEM((RB // PACK, 128), jnp.float32),
            pltpu.VMEM((32, DIM), jnp.float32),
        ],
    )
    def repack_k(stab_hbm, itab_hbm, last_hbm, ps_hbm, ph_hbm, staging,
                 pbuf, last_v):
        wid = lax.axis_index("s") * NC + lax.axis_index("c")

        def copy_rows(dst_rows):
            def rbody(i, carry):
                r0 = i * PACK
                for q in range(PACK):
                    for k in range(2):
                        v = staging[r0 + q, pl.ds(k * 16, 16)]
                        pbuf[i, pl.ds(q * DIM + k * 16, 16)] = v
                return carry
            lax.fori_loop(0, dst_rows, rbody, 0)

        def sblk(g, carry):
            bb = wid + g * NW

            @pl.when(bb < SB_FULL)
            def _():
                f = bb // FB_FULL
                b = bb % FB_FULL
                so = pl.multiple_of(b * RB, 8)
                po = pl.multiple_of(f * PSF + b * (RB // PACK), 8)
                pltpu.sync_copy(stab_hbm.at[f, pl.ds(so, RB)], staging)
                copy_rows(RB // PACK)
                pltpu.sync_copy(pbuf, ps_hbm.at[pl.ds(po, RB // PACK)])
            return carry

        lax.fori_loop(0, (SB_FULL + NW - 1) // NW, sblk, 0)

        def hblk(g, carry):
            b = wid + g * NW

            @pl.when(b < HB_FULL)
            def _():
                so = pl.multiple_of(b * RB, 8)
                po = pl.multiple_of(b * (RB // PACK), 8)
                pltpu.sync_copy(itab_hbm.at[pl.ds(so, RB)], staging)
                copy_rows(RB // PACK)
                pltpu.sync_copy(pbuf, ph_hbm.at[pl.ds(po, RB // PACK)])
            return carry

        lax.fori_loop(0, (HB_FULL + NW - 1) // NW, hblk, 0)

        @pl.when(wid < NF)
        def _stail():
            f = wid
            pltpu.sync_copy(last_hbm, last_v)
            pltpu.sync_copy(stab_hbm.at[f, pl.ds(FB_FULL * RB, 160)],
                            staging.at[pl.ds(0, 160)])
            copy_rows(40)
            for k in range(2):
                v = last_v[f, pl.ds(k * 16, 16)]
                pbuf[40, pl.ds(k * 16, 16)] = v
            pltpu.sync_copy(
                pbuf.at[pl.ds(0, 48)],
                ps_hbm.at[pl.ds(f * PSF + FB_FULL * (RB // PACK), 48)])

        @pl.when(wid == NF)
        def _htail():
            pltpu.sync_copy(last_hbm, last_v)
            pltpu.sync_copy(itab_hbm.at[pl.ds(HB_FULL * RB, 64)],
                            staging.at[pl.ds(0, 64)])
            copy_rows(16)
            for k in range(2):
                v = last_v[NF, pl.ds(k * 16, 16)]
                pbuf[16, pl.ds(k * 16, 16)] = v
            pltpu.sync_copy(pbuf.at[pl.ds(0, 24)],
                            ph_hbm.at[pl.ds(HB_FULL * (RB // PACK), 24)])

    return repack_k(stab3, itab, lastrows)


def _sc_gather(sidx, hidx, ps, ph):
    mesh = plsc.VectorSubcoreMesh(core_axis_name="c", subcore_axis_name="s")

    @functools.partial(
        pl.kernel,
        mesh=mesh,
        compiler_params=pltpu.CompilerParams(needs_layout_passes=False),
        out_type=(
            jax.ShapeDtypeStruct((S_ROWS // PACK, 128), jnp.float32),
            jax.ShapeDtypeStruct((H_ROWS // PACK, 128), jnp.float32),
        ),
        scratch_types=[
            pltpu.VMEM((S_CHUNKS, 128), jnp.int32),
            pltpu.VMEM((H_CHUNKS, 128), jnp.int32),
            pltpu.VMEM((128,), jnp.int32),
            pltpu.VMEM((128,), jnp.int32),
            pltpu.VMEM((128, 128), jnp.float32),
            pltpu.VMEM((128, 128), jnp.float32),
            pltpu.VMEM((256, 128), jnp.float32),
            pltpu.SemaphoreType.DMA,
            pltpu.SemaphoreType.DMA,
        ],
    )
    def gather_k(sidx_hbm, hidx_hbm, ps_hbm, ph_hbm, sctx_hbm, hctx_hbm,
                 sidx_v, hidx_v, pidxA, pidxB, dstA, dstB, pbuf,
                 semA, semB):
        wid = lax.axis_index("s") * NC + lax.axis_index("c")
        pltpu.sync_copy(sidx_hbm.at[wid], sidx_v)
        pltpu.sync_copy(hidx_hbm.at[wid], hidx_v)
        iota = lax.iota(jnp.int32, 16)

        slots = ((pidxA, dstA, semA), (pidxB, dstB, semB))

        def compute_pidx(idx_v, row, pidx):
            for k in range(8):
                v = idx_v[row, pl.ds(k * 16, 16)]
                pidx[pl.ds(k * 16, 16)] = lax.shift_right_logical(v, 2)

        def extract(idx_v, row, dst, c):
            def ebody(gg, carry):
                jvec = gg * 16 + iota
                idx16 = idx_v[row, pl.ds(gg * 16, 16)]
                lane0 = lax.shift_left(lax.bitwise_and(idx16, 3), 5)
                orow = c * 32 + lax.shift_right_logical(jvec, 2)
                ocol0 = lax.shift_left(lax.bitwise_and(jvec, 3), 5)
                for d in range(DIM):
                    val = plsc.load_gather(dst, [jvec, lane0 + d])
                    plsc.store_scatter(pbuf, [orow, ocol0 + d], val)
                return carry
            lax.fori_loop(0, 8, ebody, 0)

        def do_group(idx_v, tab_hbm, out_hbm, g, nchunks, out_base):
            cps = [None, None]
            for c in range(nchunks):
                pidx, dst, sem = slots[c % 2]
                row = g * 8 + c
                compute_pidx(idx_v, row, pidx)
                cpc = pltpu.async_copy(tab_hbm.at[pidx], dst, sem)
                if cps[1 - (c % 2)] is not None:
                    cps[1 - (c % 2)].wait()
                    extract(idx_v, g * 8 + c - 1, slots[(c - 1) % 2][1],
                            c - 1)
                cps[c % 2] = cpc
            last = nchunks - 1
            cps[last % 2].wait()
            extract(idx_v, g * 8 + last, slots[last % 2][1], last)
            pltpu.sync_copy(pbuf.at[pl.ds(0, nchunks * 32)],
                            out_hbm.at[pl.ds(out_base, nchunks * 32)])

        s_base = wid * (S_CHUNKS * 32)
        h_base = wid * (H_CHUNKS * 32)

        def sgrp(g, carry):
            do_group(sidx_v, ps_hbm, sctx_hbm, g, 8, s_base + g * 256)
            return carry

        lax.fori_loop(0, S_CHUNKS // 8, sgrp, 0)
        do_group(sidx_v, ps_hbm, sctx_hbm, S_CHUNKS // 8, S_CHUNKS % 8,
                 s_base + (S_CHUNKS // 8) * 256)

        def hgrp(g, carry):
            do_group(hidx_v, ph_hbm, hctx_hbm, g, 8, h_base + g * 256)
            return carry

        lax.fori_loop(0, H_CHUNKS // 8, hgrp, 0)

    return gather_k(sidx, hidx, ps, ph)


def _tc_proj_body(s_ref, h_ref, w4_ref, b4_ref, nw4_ref, seg_ref, exp_ref,
                  outs_ref, outh_ref):
    w4 = w4_ref[...]
    b4 = b4_ref[...]
    nw4 = nw4_ref[...]
    seg = seg_ref[...]
    exp = exp_ref[...]

    def proj(x):
        p = jax.lax.dot_general(
            x, w4, dimension_numbers=(((1,), (0,)), ((), ())),
            preferred_element_type=jnp.float32) + b4
        ss = jax.lax.dot_general(
            p * p, seg, dimension_numbers=(((1,), (0,)), ((), ())),
            preferred_element_type=jnp.float32) / KV_DIM
        scale = jax.lax.dot_general(
            jax.lax.rsqrt(ss + EPS), exp,
            dimension_numbers=(((1,), (0,)), ((), ())),
            preferred_element_type=jnp.float32)
        return p * scale * nw4

    outs_ref[...] = proj(s_ref[...])
    outh_ref[...] = proj(h_ref[...])


def _tc_project(sctx, hctx, w4, b4, nw4, seg, exp):
    grid = 32
    bs = S_ROWS // PACK // grid
    bh = H_ROWS // PACK // grid
    return pl.pallas_call(
        _tc_proj_body,
        grid=(grid,),
        in_specs=[
            pl.BlockSpec((bs, 128), lambda i: (i, 0)),
            pl.BlockSpec((bh, 128), lambda i: (i, 0)),
            pl.BlockSpec((128, 32), lambda i: (0, 0)),
            pl.BlockSpec((1, 32), lambda i: (0, 0)),
            pl.BlockSpec((1, 32), lambda i: (0, 0)),
            pl.BlockSpec((32, PACK), lambda i: (0, 0)),
            pl.BlockSpec((PACK, 32), lambda i: (0, 0)),
        ],
        out_specs=[
            pl.BlockSpec((bs, 32), lambda i: (i, 0)),
            pl.BlockSpec((bh, 32), lambda i: (i, 0)),
        ],
        out_shape=[
            jax.ShapeDtypeStruct((S_ROWS // PACK, 32), jnp.float32),
            jax.ShapeDtypeStruct((H_ROWS // PACK, 32), jnp.float32),
        ],
    )(sctx, hctx, w4, b4, nw4, seg, exp)


def kernel(user_sparse, history_item_ids, sparse_tables, item_table,
           W_kv, b_kv, norm_w):
    offs = (jnp.arange(NF, dtype=jnp.int32) * F_STRIDE)[None, :]
    sidx = (user_sparse + offs).reshape(NW, S_CHUNKS, 128)
    hidx = history_item_ids.reshape(NW, H_CHUNKS, 128)

    lastrows = jnp.concatenate(
        [sparse_tables[:, SV, :], item_table[IV][None, :],
         jnp.zeros((32 - NF - 1, DIM), jnp.float32)], axis=0)

    ps, ph = _sc_repack(sparse_tables, item_table, lastrows)
    sctx, hctx = _sc_gather(sidx, hidx, ps, ph)

    eye4 = jnp.eye(PACK, dtype=jnp.float32)
    w4 = jnp.einsum('pq,dk->pdqk', eye4, W_kv).reshape(PACK * DIM,
                                                       PACK * KV_DIM)
    b4 = jnp.tile(b_kv, PACK).reshape(1, PACK * KV_DIM)
    nw4 = jnp.tile(norm_w, PACK).reshape(1, PACK * KV_DIM)
    seg = jnp.repeat(jnp.eye(PACK, dtype=jnp.float32), KV_DIM, axis=0)
    exp = seg.T

    outs, outh = _tc_project(sctx, hctx, w4, b4, nw4, seg, exp)

    return jnp.concatenate(
        [outs.reshape(B, NF, KV_DIM), outh.reshape(B, HIST, KV_DIM)],
        axis=1)


# speedup vs baseline: 5.1862x; 1.0001x over previous
import functools

import jax
import jax.numpy as jnp
from jax import lax
from jax.experimental import pallas as pl
from jax.experimental.pallas import tpu as pltpu
from jax.experimental.pallas import tpu_sc as plsc

B = 4096
NF = 26
HIST = 200
DIM = 32
SV = 100000
IV = 1000000
KV_DIM = 8
EPS = 1e-6

NC = 2
NS = 16
NW = NC * NS

S_ROWS = B * NF
H_ROWS = B * HIST
S_CHUNKS = S_ROWS // (NW * 128)
H_CHUNKS = H_ROWS // (NW * 128)
PACK = 128 // DIM

IV_ROWS = IV + 1
RB = 512
FB_FULL = (SV + 1) // RB
SB_FULL = NF * FB_FULL
HB_FULL = IV_ROWS // RB
PSF = 25008
PS_ROWS = NF * PSF
PH_ROWS = 250008
F_STRIDE = PSF * PACK


def _sc_repack(stab3, itab, lastrows):
    mesh = plsc.VectorSubcoreMesh(core_axis_name="c", subcore_axis_name="s")

    @functools.partial(
        pl.kernel,
        mesh=mesh,
        compiler_params=pltpu.CompilerParams(needs_layout_passes=False),
        out_type=(
            jax.ShapeDtypeStruct((PS_ROWS, 128), jnp.float32),
            jax.ShapeDtypeStruct((PH_ROWS, 128), jnp.float32),
        ),
        scratch_types=[
            pltpu.VMEM((RB, DIM), jnp.float32),
            pltpu.VMEM((RB // PACK, 128), jnp.float32),
            pltpu.VMEM((32, DIM), jnp.float32),
        ],
    )
    def repack_k(stab_hbm, itab_hbm, last_hbm, ps_hbm, ph_hbm, staging,
                 pbuf, last_v):
        wid = lax.axis_index("s") * NC + lax.axis_index("c")

        def copy_rows(dst_rows):
            def rbody(i, carry):
                r0 = i * PACK
                for q in range(PACK):
                    for k in range(2):
                        v = staging[r0 + q, pl.ds(k * 16, 16)]
                        pbuf[i, pl.ds(q * DIM + k * 16, 16)] = v
                return carry
            lax.fori_loop(0, dst_rows, rbody, 0)

        def sblk(g, carry):
            bb = wid + g * NW

            @pl.when(bb < SB_FULL)
            def _():
                f = bb // FB_FULL
                b = bb % FB_FULL
                so = pl.multiple_of(b * RB, 8)
                po = pl.multiple_of(f * PSF + b * (RB // PACK), 8)
                pltpu.sync_copy(stab_hbm.at[f, pl.ds(so, RB)], staging)
                copy_rows(RB // PACK)
                pltpu.sync_copy(pbuf, ps_hbm.at[pl.ds(po, RB // PACK)])
            return carry

        lax.fori_loop(0, (SB_FULL + NW - 1) // NW, sblk, 0)

        def hblk(g, carry):
            b = wid + g * NW

            @pl.when(b < HB_FULL)
            def _():
                so = pl.multiple_of(b * RB, 8)
                po = pl.multiple_of(b * (RB // PACK), 8)
                pltpu.sync_copy(itab_hbm.at[pl.ds(so, RB)], staging)
                copy_rows(RB // PACK)
                pltpu.sync_copy(pbuf, ph_hbm.at[pl.ds(po, RB // PACK)])
            return carry

        lax.fori_loop(0, (HB_FULL + NW - 1) // NW, hblk, 0)

        @pl.when(wid < NF)
        def _stail():
            f = wid
            pltpu.sync_copy(last_hbm, last_v)
            pltpu.sync_copy(stab_hbm.at[f, pl.ds(FB_FULL * RB, 160)],
                            staging.at[pl.ds(0, 160)])
            copy_rows(40)
            for k in range(2):
                v = last_v[f, pl.ds(k * 16, 16)]
                pbuf[40, pl.ds(k * 16, 16)] = v
            pltpu.sync_copy(
                pbuf.at[pl.ds(0, 48)],
                ps_hbm.at[pl.ds(f * PSF + FB_FULL * (RB // PACK), 48)])

        @pl.when(wid == NF)
        def _htail():
            pltpu.sync_copy(last_hbm, last_v)
            pltpu.sync_copy(itab_hbm.at[pl.ds(HB_FULL * RB, 64)],
                            staging.at[pl.ds(0, 64)])
            copy_rows(16)
            for k in range(2):
                v = last_v[NF, pl.ds(k * 16, 16)]
                pbuf[16, pl.ds(k * 16, 16)] = v
            pltpu.sync_copy(pbuf.at[pl.ds(0, 24)],
                            ph_hbm.at[pl.ds(HB_FULL * (RB // PACK), 24)])

    return repack_k(stab3, itab, lastrows)


def _sc_gather(sidx, hidx, ps, ph):
    mesh = plsc.VectorSubcoreMesh(core_axis_name="c", subcore_axis_name="s")

    @functools.partial(
        pl.kernel,
        mesh=mesh,
        compiler_params=pltpu.CompilerParams(needs_layout_passes=False),
        out_type=(
            jax.ShapeDtypeStruct((S_ROWS // PACK, 128), jnp.float32),
            jax.ShapeDtypeStruct((H_ROWS // PACK, 128), jnp.float32),
        ),
        scratch_types=[
            pltpu.VMEM((S_CHUNKS, 128), jnp.int32),
            pltpu.VMEM((H_CHUNKS, 128), jnp.int32),
            pltpu.VMEM((128,), jnp.int32),
            pltpu.VMEM((128,), jnp.int32),
            pltpu.VMEM((128, 128), jnp.float32),
            pltpu.VMEM((128, 128), jnp.float32),
            pltpu.VMEM((256, 128), jnp.float32),
            pltpu.SemaphoreType.DMA,
            pltpu.SemaphoreType.DMA,
        ],
    )
    def gather_k(sidx_hbm, hidx_hbm, ps_hbm, ph_hbm, sctx_hbm, hctx_hbm,
                 sidx_v, hidx_v, pidxA, pidxB, dstA, dstB, pbuf,
                 semA, semB):
        wid = lax.axis_index("s") * NC + lax.axis_index("c")
        pltpu.sync_copy(sidx_hbm.at[wid], sidx_v)
        pltpu.sync_copy(hidx_hbm.at[wid], hidx_v)
        iota = lax.iota(jnp.int32, 16)

        slots = ((pidxA, dstA, semA), (pidxB, dstB, semB))

        def compute_pidx(idx_v, row, pidx):
            for k in range(8):
                v = idx_v[row, pl.ds(k * 16, 16)]
                pidx[pl.ds(k * 16, 16)] = lax.shift_right_logical(v, 2)

        def extract(idx_v, row, dst, c):
            def ebody(gg, carry):
                jvec = gg * 16 + iota
                idx16 = idx_v[row, pl.ds(gg * 16, 16)]
                lane0 = lax.shift_left(lax.bitwise_and(idx16, 3), 5)
                orow = c * 32 + lax.shift_right_logical(jvec, 2)
                ocol0 = lax.shift_left(lax.bitwise_and(jvec, 3), 5)
                for d in range(DIM):
                    val = plsc.load_gather(dst, [jvec, lane0 + d])
                    plsc.store_scatter(pbuf, [orow, ocol0 + d], val)
                return carry
            lax.fori_loop(0, 8, ebody, 0)

        def do_group(idx_v, tab_hbm, out_hbm, g, nchunks, out_base):
            cps = [None, None]
            for c in range(nchunks):
                pidx, dst, sem = slots[c % 2]
                row = g * 8 + c
                compute_pidx(idx_v, row, pidx)
                cpc = pltpu.async_copy(tab_hbm.at[pidx], dst, sem)
                if cps[1 - (c % 2)] is not None:
                    cps[1 - (c % 2)].wait()
                    extract(idx_v, g * 8 + c - 1, slots[(c - 1) % 2][1],
                            c - 1)
                cps[c % 2] = cpc
            last = nchunks - 1
            cps[last % 2].wait()
            extract(idx_v, g * 8 + last, slots[last % 2][1], last)
            pltpu.sync_copy(pbuf.at[pl.ds(0, nchunks * 32)],
                            out_hbm.at[pl.ds(out_base, nchunks * 32)])

        s_base = wid * (S_CHUNKS * 32)
        h_base = wid * (H_CHUNKS * 32)

        def sgrp(g, carry):
            do_group(sidx_v, ps_hbm, sctx_hbm, g, 8, s_base + g * 256)
            return carry

        lax.fori_loop(0, S_CHUNKS // 8, sgrp, 0)
        do_group(sidx_v, ps_hbm, sctx_hbm, S_CHUNKS // 8, S_CHUNKS % 8,
                 s_base + (S_CHUNKS // 8) * 256)

        def hgrp(g, carry):
            do_group(hidx_v, ph_hbm, hctx_hbm, g, 8, h_base + g * 256)
            return carry

        lax.fori_loop(0, H_CHUNKS // 8, hgrp, 0)

    return gather_k(sidx, hidx, ps, ph)


def _tc_proj_body(s_ref, h_ref, w4_ref, b4_ref, nw4_ref, seg_ref, exp_ref,
                  outs_ref, outh_ref):
    w4 = w4_ref[...]
    b4 = b4_ref[...]
    nw4 = nw4_ref[...]
    seg = seg_ref[...]
    exp = exp_ref[...]

    def proj(x):
        p = jax.lax.dot_general(
            x, w4, dimension_numbers=(((1,), (0,)), ((), ())),
            preferred_element_type=jnp.float32) + b4
        ss = jax.lax.dot_general(
            p * p, seg, dimension_numbers=(((1,), (0,)), ((), ())),
            preferred_element_type=jnp.float32) / KV_DIM
        scale = jax.lax.dot_general(
            jax.lax.rsqrt(ss + EPS), exp,
            dimension_numbers=(((1,), (0,)), ((), ())),
            preferred_element_type=jnp.float32)
        return p * scale * nw4

    outs_ref[...] = proj(s_ref[...])
    outh_ref[...] = proj(h_ref[...])


def _tc_project(sctx, hctx, w4, b4, nw4, seg, exp):
    grid = 32
    bs = S_ROWS // PACK // grid
    bh = H_ROWS // PACK // grid
    return pl.pallas_call(
        _tc_proj_body,
        grid=(grid,),
        in_specs=[
            pl.BlockSpec((bs, 128), lambda i: (i, 0)),
            pl.BlockSpec((bh, 128), lambda i: (i, 0)),
            pl.BlockSpec((128, 32), lambda i: (0, 0)),
            pl.BlockSpec((1, 32), lambda i: (0, 0)),
            pl.BlockSpec((1, 32), lambda i: (0, 0)),
            pl.BlockSpec((32, PACK), lambda i: (0, 0)),
            pl.BlockSpec((PACK, 32), lambda i: (0, 0)),
        ],
        out_specs=[
            pl.BlockSpec((bs, 32), lambda i: (i, 0)),
            pl.BlockSpec((bh, 32), lambda i: (i, 0)),
        ],
        out_shape=[
            jax.ShapeDtypeStruct((S_ROWS // PACK, 32), jnp.float32),
            jax.ShapeDtypeStruct((H_ROWS // PACK, 32), jnp.float32),
        ],
    )(sctx, hctx, w4, b4, nw4, seg, exp)


def kernel(user_sparse, history_item_ids, sparse_tables, item_table,
           W_kv, b_kv, norm_w):
    offs = (jnp.arange(NF, dtype=jnp.int32) * F_STRIDE)[None, :]
    sidx = (user_sparse + offs).reshape(NW, S_CHUNKS, 128)
    hidx = history_item_ids.reshape(NW, H_CHUNKS, 128)

    lastrows = jnp.concatenate(
        [sparse_tables[:, SV, :], item_table[IV][None, :],
         jnp.zeros((32 - NF - 1, DIM), jnp.float32)], axis=0)

    ps, ph = _sc_repack(sparse_tables, item_table, lastrows)
    sctx, hctx = _sc_gather(sidx, hidx, ps, ph)

    eye4 = jnp.eye(PACK, dtype=jnp.float32)
    w4 = jnp.einsum('pq,dk->pdqk', eye4, W_kv).reshape(PACK * DIM,
                                                       PACK * KV_DIM)
    b4 = jnp.tile(b_kv, PACK).reshape(1, PACK * KV_DIM)
    nw4 = jnp.tile(norm_w, PACK).reshape(1, PACK * KV_DIM)
    seg = jnp.repeat(jnp.eye(PACK, dtype=jnp.float32), KV_DIM, axis=0)
    exp = seg.T

    outs, outh = _tc_project(sctx, hctx, w4, b4, nw4, seg, exp)

    return jnp.concatenate(
        [outs.reshape(B, NF, KV_DIM), outh.reshape(B, HIST, KV_DIM)],
        axis=1)


# speedup vs baseline: 5.2131x; 1.0052x over previous
import functools

import jax
import jax.numpy as jnp
from jax import lax
from jax.experimental import pallas as pl
from jax.experimental.pallas import tpu as pltpu
from jax.experimental.pallas import tpu_sc as plsc

B = 4096
NF = 26
HIST = 200
DIM = 32
SV = 100000
IV = 1000000
KV_DIM = 8
EPS = 1e-6

NC = 2
NS = 16
NW = NC * NS

S_ROWS = B * NF
H_ROWS = B * HIST
S_CHUNKS = S_ROWS // (NW * 128)
H_CHUNKS = H_ROWS // (NW * 128)
PACK = 128 // DIM

IV_ROWS = IV + 1
RB = 256
FB_FULL = (SV + 1) // RB
SB_FULL = NF * FB_FULL
HB_FULL = IV_ROWS // RB
PSF = 25008
PS_ROWS = NF * PSF
PH_ROWS = 250008
F_STRIDE = PSF * PACK


def _sc_repack(stab3, itab, lastrows):
    mesh = plsc.VectorSubcoreMesh(core_axis_name="c", subcore_axis_name="s")

    @functools.partial(
        pl.kernel,
        mesh=mesh,
        compiler_params=pltpu.CompilerParams(needs_layout_passes=False),
        out_type=(
            jax.ShapeDtypeStruct((PS_ROWS, 128), jnp.float32),
            jax.ShapeDtypeStruct((PH_ROWS, 128), jnp.float32),
        ),
        scratch_types=[
            pltpu.VMEM((RB, DIM), jnp.float32),
            pltpu.VMEM((RB, DIM), jnp.float32),
            pltpu.VMEM((RB // PACK, 128), jnp.float32),
            pltpu.VMEM((RB // PACK, 128), jnp.float32),
            pltpu.VMEM((32, DIM), jnp.float32),
            pltpu.SemaphoreType.DMA,
            pltpu.SemaphoreType.DMA,
        ],
    )
    def repack_k(stab_hbm, itab_hbm, last_hbm, ps_hbm, ph_hbm, stagingA,
                 stagingB, pbufA, pbufB, last_v, semA, semB):
        wid = lax.axis_index("s") * NC + lax.axis_index("c")
        staging = stagingA
        pbuf = pbufA

        def copy_rows_from(stg, pb, dst_rows):
            def rbody(i, carry):
                r0 = i * PACK
                for q in range(PACK):
                    for k in range(2):
                        v = stg[r0 + q, pl.ds(k * 16, 16)]
                        pb[i, pl.ds(q * DIM + k * 16, 16)] = v
                return carry
            lax.fori_loop(0, dst_rows, rbody, 0)

        def copy_rows(dst_rows):
            copy_rows_from(stagingA, pbufA, dst_rows)

        def src_slice(is_sparse, src3, src2, bb):
            if is_sparse:
                f = bb // FB_FULL
                b = bb % FB_FULL
                so = pl.multiple_of(b * RB, 8)
                po = pl.multiple_of(f * PSF + b * (RB // PACK), 8)
                return src3.at[f, pl.ds(so, RB)], po
            so = pl.multiple_of(bb * RB, 8)
            po = pl.multiple_of(bb * (RB // PACK), 8)
            return src2.at[pl.ds(so, RB)], po

        def make_blk(is_sparse, nblocks, dst):
            def blk(g, carry):
                b0 = wid + (2 * g) * NW
                b1 = wid + (2 * g + 1) * NW

                @pl.when(b0 < nblocks)
                def _():
                    src0, po0 = src_slice(is_sparse, stab_hbm, itab_hbm, b0)
                    cp0 = pltpu.async_copy(src0, stagingA, semA)

                    @pl.when(b1 < nblocks)
                    def _():
                        src1, po1 = src_slice(is_sparse, stab_hbm,
                                              itab_hbm, b1)
                        cp1 = pltpu.async_copy(src1, stagingB, semB)
                        cp0.wait()
                        copy_rows_from(stagingA, pbufA, RB // PACK)
                        pltpu.sync_copy(pbufA,
                                        dst.at[pl.ds(po0, RB // PACK)])
                        cp1.wait()
                        copy_rows_from(stagingB, pbufB, RB // PACK)
                        pltpu.sync_copy(pbufB,
                                        dst.at[pl.ds(po1, RB // PACK)])

                    @pl.when(b1 >= nblocks)
                    def _():
                        cp0.wait()
                        copy_rows_from(stagingA, pbufA, RB // PACK)
                        pltpu.sync_copy(pbufA,
                                        dst.at[pl.ds(po0, RB // PACK)])
                return carry
            return blk

        lax.fori_loop(0, (SB_FULL + 2 * NW - 1) // (2 * NW),
                      make_blk(True, SB_FULL, ps_hbm), 0)
        lax.fori_loop(0, (HB_FULL + 2 * NW - 1) // (2 * NW),
                      make_blk(False, HB_FULL, ph_hbm), 0)

        @pl.when(wid < NF)
        def _stail():
            f = wid
            pltpu.sync_copy(last_hbm, last_v)
            pltpu.sync_copy(stab_hbm.at[f, pl.ds(FB_FULL * RB, 160)],
                            staging.at[pl.ds(0, 160)])
            copy_rows(40)
            for k in range(2):
                v = last_v[f, pl.ds(k * 16, 16)]
                pbuf[40, pl.ds(k * 16, 16)] = v
            pltpu.sync_copy(
                pbuf.at[pl.ds(0, 48)],
                ps_hbm.at[pl.ds(f * PSF + FB_FULL * (RB // PACK), 48)])

        @pl.when(wid == NF)
        def _htail():
            pltpu.sync_copy(last_hbm, last_v)
            pltpu.sync_copy(itab_hbm.at[pl.ds(HB_FULL * RB, 64)],
                            staging.at[pl.ds(0, 64)])
            copy_rows(16)
            for k in range(2):
                v = last_v[NF, pl.ds(k * 16, 16)]
                pbuf[16, pl.ds(k * 16, 16)] = v
            pltpu.sync_copy(pbuf.at[pl.ds(0, 24)],
                            ph_hbm.at[pl.ds(HB_FULL * (RB // PACK), 24)])

    return repack_k(stab3, itab, lastrows)


def _sc_gather(sidx, hidx, ps, ph):
    mesh = plsc.VectorSubcoreMesh(core_axis_name="c", subcore_axis_name="s")

    @functools.partial(
        pl.kernel,
        mesh=mesh,
        compiler_params=pltpu.CompilerParams(needs_layout_passes=False),
        out_type=(
            jax.ShapeDtypeStruct((S_ROWS // PACK, 128), jnp.float32),
            jax.ShapeDtypeStruct((H_ROWS // PACK, 128), jnp.float32),
        ),
        scratch_types=[
            pltpu.VMEM((S_CHUNKS, 128), jnp.int32),
            pltpu.VMEM((H_CHUNKS, 128), jnp.int32),
            pltpu.VMEM((128,), jnp.int32),
            pltpu.VMEM((128,), jnp.int32),
            pltpu.VMEM((128, 128), jnp.float32),
            pltpu.VMEM((128, 128), jnp.float32),
            pltpu.VMEM((256, 128), jnp.float32),
            pltpu.SemaphoreType.DMA,
            pltpu.SemaphoreType.DMA,
        ],
    )
    def gather_k(sidx_hbm, hidx_hbm, ps_hbm, ph_hbm, sctx_hbm, hctx_hbm,
                 sidx_v, hidx_v, pidxA, pidxB, dstA, dstB, pbuf,
                 semA, semB):
        wid = lax.axis_index("s") * NC + lax.axis_index("c")
        pltpu.sync_copy(sidx_hbm.at[wid], sidx_v)
        pltpu.sync_copy(hidx_hbm.at[wid], hidx_v)
        iota = lax.iota(jnp.int32, 16)

        slots = ((pidxA, dstA, semA), (pidxB, dstB, semB))

        def compute_pidx(idx_v, row, pidx):
            for k in range(8):
                v = idx_v[row, pl.ds(k * 16, 16)]
                pidx[pl.ds(k * 16, 16)] = lax.shift_right_logical(v, 2)

        def extract(idx_v, row, dst, c):
            def ebody(gg, carry):
                jvec = gg * 16 + iota
                idx16 = idx_v[row, pl.ds(gg * 16, 16)]
                lane0 = lax.shift_left(lax.bitwise_and(idx16, 3), 5)
                orow = c * 32 + lax.shift_right_logical(jvec, 2)
                ocol0 = lax.shift_left(lax.bitwise_and(jvec, 3), 5)
                for d in range(DIM):
                    val = plsc.load_gather(dst, [jvec, lane0 + d])
                    plsc.store_scatter(pbuf, [orow, ocol0 + d], val)
                return carry
            lax.fori_loop(0, 8, ebody, 0)

        def do_group(idx_v, tab_hbm, out_hbm, g, nchunks, out_base):
            cps = [None, None]
            for c in range(nchunks):
                pidx, dst, sem = slots[c % 2]
                row = g * 8 + c
                compute_pidx(idx_v, row, pidx)
                cpc = pltpu.async_copy(tab_hbm.at[pidx], dst, sem)
                if cps[1 - (c % 2)] is not None:
                    cps[1 - (c % 2)].wait()
                    extract(idx_v, g * 8 + c - 1, slots[(c - 1) % 2][1],
                            c - 1)
                cps[c % 2] = cpc
            last = nchunks - 1
            cps[last % 2].wait()
            extract(idx_v, g * 8 + last, slots[last % 2][1], last)
            pltpu.sync_copy(pbuf.at[pl.ds(0, nchunks * 32)],
                            out_hbm.at[pl.ds(out_base, nchunks * 32)])

        s_base = wid * (S_CHUNKS * 32)
        h_base = wid * (H_CHUNKS * 32)

        def sgrp(g, carry):
            do_group(sidx_v, ps_hbm, sctx_hbm, g, 8, s_base + g * 256)
            return carry

        lax.fori_loop(0, S_CHUNKS // 8, sgrp, 0)
        do_group(sidx_v, ps_hbm, sctx_hbm, S_CHUNKS // 8, S_CHUNKS % 8,
                 s_base + (S_CHUNKS // 8) * 256)

        def hgrp(g, carry):
            do_group(hidx_v, ph_hbm, hctx_hbm, g, 8, h_base + g * 256)
            return carry

        lax.fori_loop(0, H_CHUNKS // 8, hgrp, 0)

    return gather_k(sidx, hidx, ps, ph)


def _tc_proj_body(s_ref, h_ref, w4_ref, b4_ref, nw4_ref, seg_ref, exp_ref,
                  outs_ref, outh_ref):
    w4 = w4_ref[...]
    b4 = b4_ref[...]
    nw4 = nw4_ref[...]
    seg = seg_ref[...]
    exp = exp_ref[...]

    def proj(x):
        p = jax.lax.dot_general(
            x, w4, dimension_numbers=(((1,), (0,)), ((), ())),
            preferred_element_type=jnp.float32) + b4
        ss = jax.lax.dot_general(
            p * p, seg, dimension_numbers=(((1,), (0,)), ((), ())),
            preferred_element_type=jnp.float32) / KV_DIM
        scale = jax.lax.dot_general(
            jax.lax.rsqrt(ss + EPS), exp,
            dimension_numbers=(((1,), (0,)), ((), ())),
            preferred_element_type=jnp.float32)
        return p * scale * nw4

    outs_ref[...] = proj(s_ref[...])
    outh_ref[...] = proj(h_ref[...])


def _tc_project(sctx, hctx, w4, b4, nw4, seg, exp):
    grid = 32
    bs = S_ROWS // PACK // grid
    bh = H_ROWS // PACK // grid
    return pl.pallas_call(
        _tc_proj_body,
        grid=(grid,),
        in_specs=[
            pl.BlockSpec((bs, 128), lambda i: (i, 0)),
            pl.BlockSpec((bh, 128), lambda i: (i, 0)),
            pl.BlockSpec((128, 32), lambda i: (0, 0)),
            pl.BlockSpec((1, 32), lambda i: (0, 0)),
            pl.BlockSpec((1, 32), lambda i: (0, 0)),
            pl.BlockSpec((32, PACK), lambda i: (0, 0)),
            pl.BlockSpec((PACK, 32), lambda i: (0, 0)),
        ],
        out_specs=[
            pl.BlockSpec((bs, 32), lambda i: (i, 0)),
            pl.BlockSpec((bh, 32), lambda i: (i, 0)),
        ],
        out_shape=[
            jax.ShapeDtypeStruct((S_ROWS // PACK, 32), jnp.float32),
            jax.ShapeDtypeStruct((H_ROWS // PACK, 32), jnp.float32),
        ],
    )(sctx, hctx, w4, b4, nw4, seg, exp)


def kernel(user_sparse, history_item_ids, sparse_tables, item_table,
           W_kv, b_kv, norm_w):
    offs = (jnp.arange(NF, dtype=jnp.int32) * F_STRIDE)[None, :]
    sidx = (user_sparse + offs).reshape(NW, S_CHUNKS, 128)
    hidx = history_item_ids.reshape(NW, H_CHUNKS, 128)

    lastrows = jnp.concatenate(
        [sparse_tables[:, SV, :], item_table[IV][None, :],
         jnp.zeros((32 - NF - 1, DIM), jnp.float32)], axis=0)

    ps, ph = _sc_repack(sparse_tables, item_table, lastrows)
    sctx, hctx = _sc_gather(sidx, hidx, ps, ph)

    eye4 = jnp.eye(PACK, dtype=jnp.float32)
    w4 = jnp.einsum('pq,dk->pdqk', eye4, W_kv).reshape(PACK * DIM,
                                                       PACK * KV_DIM)
    b4 = jnp.tile(b_kv, PACK).reshape(1, PACK * KV_DIM)
    nw4 = jnp.tile(norm_w, PACK).reshape(1, PACK * KV_DIM)
    seg = jnp.repeat(jnp.eye(PACK, dtype=jnp.float32), KV_DIM, axis=0)
    exp = seg.T

    outs, outh = _tc_project(sctx, hctx, w4, b4, nw4, seg, exp)

    return jnp.concatenate(
        [outs.reshape(B, NF, KV_DIM), outh.reshape(B, HIST, KV_DIM)],
        axis=1)
